# baseline (speedup 1.0000x reference)
import functools

import numpy as np
import jax
import jax.numpy as jnp
from jax import lax
from jax.experimental import pallas as pl
from jax.experimental.pallas import tpu as pltpu
from jax.experimental.pallas import tpu_sc as plsc

DIM = 512
HALF = DIM // 2
ROWS_BLK = 256
N_HI = 27
HI_PER_BLK = 3
NROWS_PAD = N_HI * ROWS_BLK
B_TOK = 4 * 8192


def _pow3_f32(k):
    out = jnp.ones_like(k, dtype=jnp.float32)
    for i in range(7, 0, -1):
        out = jnp.where(k == i, np.float32(3.0 ** i), out)
    return out


def _codes_body(m_ref, codes_ref):
    m = m_ref[0]
    lo = m[0:1, :]
    for k in range(1, 5):
        lo = lo + m[k:k + 1, :] * (3 ** k)
    hi = m[5:6, :] + m[6:7, :] * 3 + m[7:8, :] * 9
    codes = lo + hi * 256
    codes_ref[...] = jnp.reshape(codes, (64, 128))


def _table_body(at_ref, ct_ref, ft_ref, dt_ref, nt_ref, rw_ref, pt_ref,
                ps_ref, t_ref, tlow_ref, thigh_ref):
    pid = pl.program_id(0)

    @pl.when(pid == 0)
    def _build_parts():
        rr = lax.broadcasted_iota(jnp.int32, (8, 16), 0) + 2 ** 15
        jj = lax.broadcasted_iota(jnp.int32, (8, 16), 1)
        bits = ((rr >> jj) & 1).astype(jnp.float32)
        r3 = lax.dot_general(bits, rw_ref[...], (((1,), (1,)), ((), ())),
                             preferred_element_type=jnp.float32)

        ps = ps_ref[0]
        kk = lax.broadcasted_iota(jnp.int32, (1, HALF), 1).astype(jnp.float32)
        rho = jnp.exp(np.float32(np.log(10.0)) * kk *
                      np.float32(1.0 / (HALF - 1)))
        wx = rho * jnp.cos(kk)
        wy = rho * jnp.sin(kk)
        j9 = lax.broadcasted_iota(jnp.int32, (16, 1), 0)
        h9 = (j9 // 3).astype(jnp.float32) * np.float32(1e-5)
        w9 = (j9 % 3).astype(jnp.float32) * np.float32(1e-5)
        phase = np.float32(np.pi) * (wx * h9 + wy * w9)
        pos9 = jnp.concatenate([jnp.cos(phase), jnp.sin(phase)], axis=1) * ps

        comp = jnp.concatenate(
            [at_ref[0:3, :], ct_ref[0:3, :], ft_ref[0:3, :], dt_ref[0:3, :],
             nt_ref[0:3, :], r3[0:3, :], pos9[0:9, :], pt_ref[0:1, :],
             jnp.zeros((4, DIM), jnp.float32)], axis=0)

        comp_h = comp.astype(jnp.bfloat16).astype(jnp.float32)
        comp2 = jnp.concatenate(
            [comp_h, comp - comp_h], axis=0).astype(jnp.bfloat16)

        lane = lax.broadcasted_iota(jnp.int32, (ROWS_BLK, 64), 1)
        lane = jnp.where(lane >= 32, lane - 32, lane)

        rl = lax.broadcasted_iota(jnp.int32, (ROWS_BLK, 1), 0)
        d0 = rl % 3
        d1 = (rl // 3) % 3
        d2 = (rl // 9) % 3
        d3 = (rl // 27) % 3
        d4 = (rl // 81) % 3
        ohl = (lane == d0).astype(jnp.int32)
        ohl = ohl + (lane == d3 + 3).astype(jnp.int32)
        ohl = ohl + (lane == d4 + 6).astype(jnp.int32)
        ohl = ohl + (lane == d1 * 3 + d2 + 18).astype(jnp.int32)
        tlow_ref[...] = lax.dot_general(
            ohl.astype(jnp.bfloat16), comp2, (((1,), (0,)), ((), ())),
            preferred_element_type=jnp.float32)

        rh = lax.broadcasted_iota(jnp.int32, (32, 1), 0)
        e0 = rh % 3
        e1 = (rh // 3) % 3
        e2 = (rh // 9) % 3
        lane_h = lane[0:32, :]
        ohh = (lane_h == e0 + 9).astype(jnp.int32)
        ohh = ohh + (lane_h == e1 + 12).astype(jnp.int32)
        ohh = ohh + (lane_h == e2 + 15).astype(jnp.int32)
        thigh_ref[...] = lax.dot_general(
            ohh.astype(jnp.bfloat16), comp2, (((1,), (0,)), ((), ())),
            preferred_element_type=jnp.float32)

    tlow = tlow_ref[...]
    for j in range(HI_PER_BLK):
        t_ref[j * ROWS_BLK:(j + 1) * ROWS_BLK, :] = (
            tlow + thigh_ref[pl.ds(pid * HI_PER_BLK + j, 1), :])

    @pl.when(pid == 0)
    def _pad_row():
        t_ref[0:1, :] = pt_ref[...]


def _build_codes(metadata_ids):
    m_t = jnp.transpose(metadata_ids, (0, 2, 1))
    codes = pl.pallas_call(
        _codes_body,
        grid=(4,),
        in_specs=[pl.BlockSpec((1, 8, 8192), lambda i: (i, 0, 0))],
        out_specs=pl.BlockSpec((64, 128), lambda i: (i, 0)),
        out_shape=jax.ShapeDtypeStruct((256, 128), jnp.int32),
    )(m_t)
    return codes.reshape(B_TOK)


def _build_table(action_table, channel_table, filter_table, depth_table,
                 n_table, rec_W, pad_token, pos_scale):
    full = lambda s: pl.BlockSpec(s, lambda i: (0, 0))
    return pl.pallas_call(
        _table_body,
        grid=(N_HI // HI_PER_BLK,),
        in_specs=[
            full((8, DIM)), full((3, DIM)), full((4, DIM)), full((12, DIM)),
            full((16, DIM)), full((DIM, 16)), full((1, DIM)),
            pl.BlockSpec(memory_space=pltpu.SMEM),
        ],
        out_specs=pl.BlockSpec((ROWS_BLK * HI_PER_BLK, DIM), lambda i: (i, 0)),
        out_shape=jax.ShapeDtypeStruct((NROWS_PAD, DIM), jnp.float32),
        scratch_shapes=[pltpu.VMEM((ROWS_BLK, DIM), jnp.float32),
                        pltpu.VMEM((32, DIM), jnp.float32)],
    )(action_table, channel_table, filter_table, depth_table, n_table,
      rec_W, pad_token, pos_scale.reshape(1))


def _sc_gather(table, codes):
    info = plsc.get_sparse_core_info()
    nw = info.num_cores * info.num_subcores
    per_w = B_TOK // nw
    ch = 32
    n_ch = per_w // ch
    nbuf = 6
    lead = 3
    mesh = plsc.VectorSubcoreMesh(core_axis_name="c", subcore_axis_name="s")

    @functools.partial(
        pl.kernel,
        mesh=mesh,
        out_type=jax.ShapeDtypeStruct((B_TOK, DIM), jnp.float32),
        scratch_types=(
            [pltpu.VMEM((per_w,), jnp.int32)] +
            [pltpu.VMEM((ch, DIM), jnp.float32)] * 6 +
            [pltpu.SemaphoreType.DMA] * 12
        ),
    )
    def k(t_hbm, codes_hbm, out_hbm, idx_v, b0, b1, b2, b3, b4, b5,
          g0, g1, g2, g3, g4, g5, s0, s1, s2, s3, s4, s5):
        wid = lax.axis_index("s") * info.num_cores + lax.axis_index("c")
        base = wid * per_w
        bufs = (b0, b1, b2, b3, b4, b5)
        gsems = (g0, g1, g2, g3, g4, g5)
        ssems = (s0, s1, s2, s3, s4, s5)
        pltpu.sync_copy(codes_hbm.at[pl.ds(base, per_w)], idx_v)

        def gather(c, p):
            return pltpu.async_copy(
                t_hbm.at[idx_v.at[pl.ds(c * ch, ch)]], bufs[p], gsems[p])

        def wait_gather(c, p):
            pltpu.make_async_copy(
                t_hbm.at[idx_v.at[pl.ds(c * ch, ch)]], bufs[p],
                gsems[p]).wait()

        def store(c, p):
            return pltpu.async_copy(
                bufs[p], out_hbm.at[pl.ds(base + c * ch, ch)], ssems[p])

        def wait_store(c, p):
            pltpu.make_async_copy(
                bufs[p], out_hbm.at[pl.ds(base + c * ch, ch)], ssems[p]).wait()

        def slot(cc, p, refill=True):
            wait_gather(cc, p)
            store(cc, p)
            if not refill:
                return
            nxt = cc + lead
            pn = (p + lead) % nbuf

            @pl.when(nxt < n_ch)
            def _refill():
                @pl.when(nxt >= nbuf)
                def _w():
                    wait_store(nxt - nbuf, pn)
                gather(nxt, pn)

        for p in range(lead):
            gather(p, p)

        def body(i, carry):
            c = i * nbuf
            for p in range(nbuf):
                slot(c + p, p)
            return carry

        lax.fori_loop(0, n_ch // nbuf, body, 0)
        for p in range(n_ch - (n_ch // nbuf) * nbuf):
            cc = (n_ch // nbuf) * nbuf + p
            slot(cc, cc % nbuf, refill=cc + lead < n_ch)
        for cc in range(n_ch - nbuf, n_ch):
            wait_store(cc, cc % nbuf)

    return k(table, codes)


def kernel(metadata_ids, action_table, channel_table, filter_table,
           depth_table, n_table, rec_W, pad_token, pos_scale):
    codes = _build_codes(metadata_ids)
    table = _build_table(action_table, channel_table, filter_table,
                         depth_table, n_table, rec_W, pad_token, pos_scale)
    out = _sc_gather(table, codes)
    return out.reshape(metadata_ids.shape[0], metadata_ids.shape[1], DIM)


# speedup vs baseline: 1.0017x; 1.0017x over previous
import functools

import numpy as np
import jax
import jax.numpy as jnp
from jax import lax
from jax.experimental import pallas as pl
from jax.experimental.pallas import tpu as pltpu
from jax.experimental.pallas import tpu_sc as plsc

DIM = 512
HALF = DIM // 2
ROWS_BLK = 256
N_HI = 27
HI_PER_BLK = 3
NROWS_PAD = N_HI * ROWS_BLK
B_TOK = 4 * 8192


def _pow3_f32(k):
    out = jnp.ones_like(k, dtype=jnp.float32)
    for i in range(7, 0, -1):
        out = jnp.where(k == i, np.float32(3.0 ** i), out)
    return out


def _codes_body(m_ref, codes_ref):
    m = m_ref[0]
    lo = m[0:1, :]
    for k in range(1, 5):
        lo = lo + m[k:k + 1, :] * (3 ** k)
    hi = m[5:6, :] + m[6:7, :] * 3 + m[7:8, :] * 9
    codes = lo + hi * 256
    codes_ref[...] = jnp.reshape(codes, (64, 128))


def _table_body(at_ref, ct_ref, ft_ref, dt_ref, nt_ref, rw_ref, pt_ref,
                ps_ref, t_ref, tlow_ref, thigh_ref):
    pid = pl.program_id(0)

    @pl.when(pid == 0)
    def _build_parts():
        rr = lax.broadcasted_iota(jnp.int32, (8, 16), 0) + 2 ** 15
        jj = lax.broadcasted_iota(jnp.int32, (8, 16), 1)
        bits = ((rr >> jj) & 1).astype(jnp.float32)
        r3 = lax.dot_general(bits, rw_ref[...], (((1,), (1,)), ((), ())),
                             preferred_element_type=jnp.float32)

        ps = ps_ref[0]
        kk = lax.broadcasted_iota(jnp.int32, (1, HALF), 1).astype(jnp.float32)
        rho = jnp.exp(np.float32(np.log(10.0)) * kk *
                      np.float32(1.0 / (HALF - 1)))
        wx = rho * jnp.cos(kk)
        wy = rho * jnp.sin(kk)
        j9 = lax.broadcasted_iota(jnp.int32, (16, 1), 0)
        h9 = (j9 // 3).astype(jnp.float32) * np.float32(1e-5)
        w9 = (j9 % 3).astype(jnp.float32) * np.float32(1e-5)
        phase = np.float32(np.pi) * (wx * h9 + wy * w9)
        pos9 = jnp.concatenate([jnp.cos(phase), jnp.sin(phase)], axis=1) * ps

        comp = jnp.concatenate(
            [at_ref[0:3, :], ct_ref[0:3, :], ft_ref[0:3, :], dt_ref[0:3, :],
             nt_ref[0:3, :], r3[0:3, :], pos9[0:9, :], pt_ref[0:1, :],
             jnp.zeros((4, DIM), jnp.float32)], axis=0)

        comp_h = comp.astype(jnp.bfloat16).astype(jnp.float32)
        comp2 = jnp.concatenate(
            [comp_h, comp - comp_h], axis=0).astype(jnp.bfloat16)

        lane = lax.broadcasted_iota(jnp.int32, (ROWS_BLK, 64), 1)
        lane = jnp.where(lane >= 32, lane - 32, lane)

        rl = lax.broadcasted_iota(jnp.int32, (ROWS_BLK, 1), 0)
        d0 = rl % 3
        d1 = (rl // 3) % 3
        d2 = (rl // 9) % 3
        d3 = (rl // 27) % 3
        d4 = (rl // 81) % 3
        ohl = (lane == d0).astype(jnp.int32)
        ohl = ohl + (lane == d3 + 3).astype(jnp.int32)
        ohl = ohl + (lane == d4 + 6).astype(jnp.int32)
        ohl = ohl + (lane == d1 * 3 + d2 + 18).astype(jnp.int32)
        tlow_ref[...] = lax.dot_general(
            ohl.astype(jnp.bfloat16), comp2, (((1,), (0,)), ((), ())),
            preferred_element_type=jnp.float32)

        rh = lax.broadcasted_iota(jnp.int32, (32, 1), 0)
        e0 = rh % 3
        e1 = (rh // 3) % 3
        e2 = (rh // 9) % 3
        lane_h = lane[0:32, :]
        ohh = (lane_h == e0 + 9).astype(jnp.int32)
        ohh = ohh + (lane_h == e1 + 12).astype(jnp.int32)
        ohh = ohh + (lane_h == e2 + 15).astype(jnp.int32)
        thigh_ref[...] = lax.dot_general(
            ohh.astype(jnp.bfloat16), comp2, (((1,), (0,)), ((), ())),
            preferred_element_type=jnp.float32)

    tlow = tlow_ref[...]
    for j in range(HI_PER_BLK):
        t_ref[j * ROWS_BLK:(j + 1) * ROWS_BLK, :] = (
            tlow + thigh_ref[pl.ds(pid * HI_PER_BLK + j, 1), :])

    @pl.when(pid == 0)
    def _pad_row():
        t_ref[0:1, :] = pt_ref[...]


def _build_codes(metadata_ids):
    m_t = jnp.transpose(metadata_ids, (0, 2, 1))
    codes = pl.pallas_call(
        _codes_body,
        grid=(4,),
        in_specs=[pl.BlockSpec((1, 8, 8192), lambda i: (i, 0, 0))],
        out_specs=pl.BlockSpec((64, 128), lambda i: (i, 0)),
        out_shape=jax.ShapeDtypeStruct((256, 128), jnp.int32),
    )(m_t)
    return codes.reshape(B_TOK)


def _build_table(action_table, channel_table, filter_table, depth_table,
                 n_table, rec_W, pad_token, pos_scale):
    full = lambda s: pl.BlockSpec(s, lambda i: (0, 0))
    return pl.pallas_call(
        _table_body,
        grid=(N_HI // HI_PER_BLK,),
        in_specs=[
            full((8, DIM)), full((3, DIM)), full((4, DIM)), full((12, DIM)),
            full((16, DIM)), full((DIM, 16)), full((1, DIM)),
            pl.BlockSpec(memory_space=pltpu.SMEM),
        ],
        out_specs=pl.BlockSpec((ROWS_BLK * HI_PER_BLK, DIM), lambda i: (i, 0)),
        out_shape=jax.ShapeDtypeStruct((NROWS_PAD, DIM), jnp.float32),
        scratch_shapes=[pltpu.VMEM((ROWS_BLK, DIM), jnp.float32),
                        pltpu.VMEM((32, DIM), jnp.float32)],
    )(action_table, channel_table, filter_table, depth_table, n_table,
      rec_W, pad_token, pos_scale.reshape(1))


def _sc_gather(table, codes):
    info = plsc.get_sparse_core_info()
    nw = info.num_cores * info.num_subcores
    per_w = B_TOK // nw
    ch = 64
    n_ch = per_w // ch
    nbuf = 3
    mesh = plsc.VectorSubcoreMesh(core_axis_name="c", subcore_axis_name="s")

    @functools.partial(
        pl.kernel,
        mesh=mesh,
        out_type=jax.ShapeDtypeStruct((B_TOK, DIM), jnp.float32),
        scratch_types=[
            pltpu.VMEM((per_w,), jnp.int32),
            pltpu.VMEM((ch, DIM), jnp.float32),
            pltpu.VMEM((ch, DIM), jnp.float32),
            pltpu.VMEM((ch, DIM), jnp.float32),
            pltpu.SemaphoreType.DMA,
            pltpu.SemaphoreType.DMA,
            pltpu.SemaphoreType.DMA,
            pltpu.SemaphoreType.DMA,
            pltpu.SemaphoreType.DMA,
            pltpu.SemaphoreType.DMA,
        ],
    )
    def k(t_hbm, codes_hbm, out_hbm, idx_v, buf0, buf1, buf2,
          g0, g1, g2, s0, s1, s2):
        wid = lax.axis_index("s") * info.num_cores + lax.axis_index("c")
        base = wid * per_w
        bufs = (buf0, buf1, buf2)
        gsems = (g0, g1, g2)
        ssems = (s0, s1, s2)
        pltpu.sync_copy(codes_hbm.at[pl.ds(base, per_w)], idx_v)

        def gather(c, p):
            return pltpu.async_copy(
                t_hbm.at[idx_v.at[pl.ds(c * ch, ch)]], bufs[p], gsems[p])

        def store(c, p):
            return pltpu.async_copy(
                bufs[p], out_hbm.at[pl.ds(base + c * ch, ch)], ssems[p])

        for p in range(nbuf):
            gather(p, p)

        def body(i, carry):
            c = i * nbuf
            for p in range(nbuf):
                cc = c + p
                pltpu.make_async_copy(
                    t_hbm.at[idx_v.at[pl.ds(cc * ch, ch)]], bufs[p],
                    gsems[p]).wait()
                store(cc, p)

                @pl.when(cc + nbuf < n_ch)
                def _():
                    pltpu.make_async_copy(
                        bufs[p], out_hbm.at[pl.ds(base + cc * ch, ch)],
                        ssems[p]).wait()
                    gather(cc + nbuf, p)

            return carry

        lax.fori_loop(0, n_ch // nbuf, body, 0)
        cc = (n_ch // nbuf) * nbuf
        pltpu.make_async_copy(
            t_hbm.at[idx_v.at[pl.ds(cc * ch, ch)]], bufs[0], gsems[0]).wait()
        store(cc, 0)
        pltpu.make_async_copy(
            buf1, out_hbm.at[pl.ds(base + 13 * ch, ch)], s1).wait()
        pltpu.make_async_copy(
            buf2, out_hbm.at[pl.ds(base + 14 * ch, ch)], s2).wait()
        pltpu.make_async_copy(
            buf0, out_hbm.at[pl.ds(base + 15 * ch, ch)], s0).wait()

    return k(table, codes)


def kernel(metadata_ids, action_table, channel_table, filter_table,
           depth_table, n_table, rec_W, pad_token, pos_scale):
    codes = _build_codes(metadata_ids)
    table = _build_table(action_table, channel_table, filter_table,
                         depth_table, n_table, rec_W, pad_token, pos_scale)
    out = _sc_gather(table, codes)
    return out.reshape(metadata_ids.shape[0], metadata_ids.shape[1], DIM)
